# baseline (speedup 1.0000x reference)
import functools

import jax
import jax.numpy as jnp
from jax import lax
from jax.experimental import pallas as pl
from jax.experimental.pallas import tpu as pltpu
from jax.experimental.pallas import tpu_sc as plsc

DEPTH = 1000
B0, B1 = 1024, 26
NC, NS = 2, 16
NW = NC * NS
COLS = 128
NCHUNK = B1 * (B0 // COLS)
NT = -(-NCHUNK // NW)
HALVES = ((0, 496), (496, 504))


def _one_hot_sc(xt_flat, zero_half):
    mesh = plsc.VectorSubcoreMesh(core_axis_name="c", subcore_axis_name="s")

    @functools.partial(
        pl.kernel,
        mesh=mesh,
        out_type=jax.ShapeDtypeStruct((B1, DEPTH, B0), jnp.int32),
        compiler_params=pltpu.CompilerParams(needs_layout_passes=False),
        scratch_types=[
            pltpu.VMEM((NT * COLS,), jnp.int32),
            pltpu.VMEM((HALVES[0][1], COLS), jnp.int32),
            pltpu.VMEM((HALVES[1][1], COLS), jnp.int32),
            pltpu.SemaphoreType.DMA,
            pltpu.SemaphoreType.DMA,
            pltpu.SemaphoreType.DMA,
        ],
    )
    def k(xt_hbm, z_hbm, out_hbm, xv, buf_a, buf_b, sem_a, sem_b, sem_x):
        wid = lax.axis_index("s") * NC + lax.axis_index("c")
        bufs = (buf_a, buf_b)
        sems = (sem_a, sem_b)

        for t in range(NT):
            g = t * NW + wid

            @pl.when(g < NCHUNK)
            def _():
                pltpu.async_copy(xt_hbm.at[pl.ds(g * COLS, COLS)],
                                 xv.at[pl.ds(t * COLS, COLS)], sem_x)

        for h, (r0, nr) in enumerate(HALVES):
            pltpu.async_copy(z_hbm.at[pl.ds(0, nr)], bufs[h], sems[h])

        for t in range(NT):
            g = t * NW + wid

            @pl.when(g < NCHUNK)
            def _():
                pltpu.make_async_copy(xt_hbm.at[pl.ds(0, COLS)],
                                      xv.at[pl.ds(0, COLS)], sem_x).wait()

        lane = lax.iota(jnp.int32, 16)
        ones = jnp.full((16,), 1, jnp.int32)
        zeros = jnp.zeros((16,), jnp.int32)

        def scatter_half(h, t, vals):
            r0, nr = HALVES[h]
            for s in range(COLS // 16):
                rows = xv[pl.ds(t * COLS + s * 16, 16)]
                mask = (rows >= r0) & (rows < r0 + nr)
                plsc.store_scatter(bufs[h], [rows - r0, lane + s * 16],
                                   vals, mask=mask)

        for t in range(NT):
            g = t * NW + wid
            j = g // (B0 // COLS)
            c = g % (B0 // COLS)

            @pl.when(g < NCHUNK)
            def _():
                for h, (r0, nr) in enumerate(HALVES):
                    pltpu.make_async_copy(
                        bufs[h], out_hbm.at[0, pl.ds(r0, nr), pl.ds(0, COLS)],
                        sems[h]).wait()
                    if t > 0:
                        scatter_half(h, t - 1, zeros)
                    scatter_half(h, t, ones)
                    pltpu.async_copy(
                        bufs[h],
                        out_hbm.at[j, pl.ds(r0, nr), pl.ds(c * COLS, COLS)],
                        sems[h])

        for h, (r0, nr) in enumerate(HALVES):
            pltpu.make_async_copy(
                bufs[h], out_hbm.at[0, pl.ds(r0, nr), pl.ds(0, COLS)],
                sems[h]).wait()

    return k(xt_flat, zero_half)


def kernel(x):
    xt_flat = x.T.reshape(-1)
    zero_half = jnp.zeros((HALVES[1][1], COLS), jnp.int32)
    out_t = _one_hot_sc(xt_flat, zero_half)
    return jnp.transpose(out_t, (2, 0, 1))


# speedup vs baseline: 1.2641x; 1.2641x over previous
import functools

import jax
import jax.numpy as jnp
from jax import lax
from jax.experimental import pallas as pl
from jax.experimental.pallas import tpu as pltpu
from jax.experimental.pallas import tpu_sc as plsc

DEPTH = 1000
B0, B1 = 1024, 26
NC, NS = 2, 16
NW = NC * NS
COLS = 128
NCHUNK = B1 * (B0 // COLS)
NT = -(-NCHUNK // NW)


def _one_hot_sc(xt_flat, zero_chunk):
    mesh = plsc.VectorSubcoreMesh(core_axis_name="c", subcore_axis_name="s")

    @functools.partial(
        pl.kernel,
        mesh=mesh,
        out_type=jax.ShapeDtypeStruct((B1, DEPTH, B0), jnp.int32),
        compiler_params=pltpu.CompilerParams(needs_layout_passes=False),
        scratch_types=[
            pltpu.VMEM((NT * COLS,), jnp.int32),
            pltpu.VMEM((DEPTH, COLS), jnp.int32),
            pltpu.SemaphoreType.DMA,
            pltpu.SemaphoreType.DMA,
        ],
    )
    def k(xt_hbm, z_hbm, out_hbm, xv, buf, sem, sem_x):
        wid = lax.axis_index("s") * NC + lax.axis_index("c")

        for t in range(NT):
            g = t * NW + wid

            @pl.when(g < NCHUNK)
            def _():
                pltpu.async_copy(xt_hbm.at[pl.ds(g * COLS, COLS)],
                                 xv.at[pl.ds(t * COLS, COLS)], sem_x)

        pltpu.async_copy(z_hbm, buf, sem)

        for t in range(NT):
            g = t * NW + wid

            @pl.when(g < NCHUNK)
            def _():
                pltpu.make_async_copy(xt_hbm.at[pl.ds(0, COLS)],
                                      xv.at[pl.ds(0, COLS)], sem_x).wait()

        lane = lax.iota(jnp.int32, 16)
        ones = jnp.full((16,), 1, jnp.int32)
        zeros = jnp.zeros((16,), jnp.int32)

        def scatter_chunk(t, vals):
            for s in range(COLS // 16):
                rows = xv[pl.ds(t * COLS + s * 16, 16)]
                plsc.store_scatter(buf, [rows, lane + s * 16], vals)

        for t in range(NT):
            g = t * NW + wid
            j = g // (B0 // COLS)
            c = g % (B0 // COLS)

            @pl.when(g < NCHUNK)
            def _():
                pltpu.make_async_copy(
                    buf, out_hbm.at[0, :, pl.ds(0, COLS)], sem).wait()
                if t > 0:
                    scatter_chunk(t - 1, zeros)
                scatter_chunk(t, ones)
                pltpu.async_copy(
                    buf, out_hbm.at[j, :, pl.ds(c * COLS, COLS)], sem)

        pltpu.make_async_copy(buf, out_hbm.at[0, :, pl.ds(0, COLS)],
                              sem).wait()

    return k(xt_flat, zero_chunk)


def kernel(x):
    xt_flat = x.T.reshape(-1)
    zero_chunk = jnp.zeros((DEPTH, COLS), jnp.int32)
    out_t = _one_hot_sc(xt_flat, zero_chunk)
    return jnp.transpose(out_t, (2, 0, 1))


# speedup vs baseline: 1.3272x; 1.0499x over previous
import functools

import jax
import jax.numpy as jnp
from jax import lax
from jax.experimental import pallas as pl
from jax.experimental.pallas import tpu as pltpu
from jax.experimental.pallas import tpu_sc as plsc

DEPTH = 1000
B0, B1 = 1024, 26
NC, NS = 2, 16
NW = NC * NS
COLS = 128
NCHUNK = B1 * (B0 // COLS)
NT = -(-NCHUNK // NW)


def _one_hot_sc(xt_flat, zero_chunk):
    mesh = plsc.VectorSubcoreMesh(core_axis_name="c", subcore_axis_name="s")

    @functools.partial(
        pl.kernel,
        mesh=mesh,
        out_type=jax.ShapeDtypeStruct((B1, DEPTH, B0), jnp.int32),
        compiler_params=pltpu.CompilerParams(needs_layout_passes=False),
        scratch_types=[
            pltpu.VMEM((NT * COLS,), jnp.int32),
            pltpu.VMEM((DEPTH, COLS), jnp.int32),
            pltpu.SemaphoreType.DMA,
            pltpu.SemaphoreType.DMA,
        ],
    )
    def k(xt_hbm, z_hbm, out_hbm, xv, buf, sem, sem_x):
        wid = lax.axis_index("s") * NC + lax.axis_index("c")
        g_last = (NT - 1) * NW + (wid & (NW // 2 - 1))

        def chunk_id(t):
            return g_last if t == NT - 1 else t * NW + wid

        for t in range(NT):
            pltpu.async_copy(xt_hbm.at[pl.ds(chunk_id(t) * COLS, COLS)],
                             xv.at[pl.ds(t * COLS, COLS)], sem_x)

        pltpu.async_copy(z_hbm, buf, sem)

        for t in range(NT):
            pltpu.make_async_copy(xt_hbm.at[pl.ds(0, COLS)],
                                  xv.at[pl.ds(0, COLS)], sem_x).wait()

        lane = lax.iota(jnp.int32, 16)
        ones = jnp.full((16,), 1, jnp.int32)
        zeros = jnp.zeros((16,), jnp.int32)
        upper = wid >= NW // 2

        def scatter_chunk(t, vals, mask=None):
            for s in range(COLS // 16):
                rows = xv[pl.ds(t * COLS + s * 16, 16)]
                m = None if mask is None else mask(rows)
                plsc.store_scatter(buf, [rows, lane + s * 16], vals, mask=m)

        for t in range(NT):
            g = chunk_id(t)
            j = g // (B0 // COLS)
            c = g % (B0 // COLS)

            pltpu.make_async_copy(
                buf, out_hbm.at[0, :, pl.ds(0, COLS)], sem).wait()
            if t > 0:
                scatter_chunk(t - 1, zeros)
            if t < NT - 1:
                scatter_chunk(t, ones)
                pltpu.async_copy(
                    buf, out_hbm.at[j, :, pl.ds(c * COLS, COLS)], sem)
            else:
                for r0, nr, cond in ((0, 496, ~upper), (496, 504, upper)):
                    @pl.when(cond)
                    def _(r0=r0, nr=nr):
                        scatter_chunk(
                            t, ones,
                            mask=lambda rows: ((rows >= r0) & (rows < r0 + nr)))
                        pltpu.async_copy(
                            buf.at[pl.ds(r0, nr)],
                            out_hbm.at[j, pl.ds(r0, nr),
                                       pl.ds(c * COLS, COLS)],
                            sem)

        for r0, nr, cond in ((0, 496, ~upper), (496, 504, upper)):
            @pl.when(cond)
            def _(r0=r0, nr=nr):
                pltpu.make_async_copy(
                    buf.at[pl.ds(r0, nr)],
                    out_hbm.at[0, pl.ds(r0, nr), pl.ds(0, COLS)], sem).wait()

    return k(xt_flat, zero_chunk)


def kernel(x):
    xt_flat = x.T.reshape(-1)
    zero_chunk = jnp.zeros((DEPTH, COLS), jnp.int32)
    out_t = _one_hot_sc(xt_flat, zero_chunk)
    return jnp.transpose(out_t, (2, 0, 1))
